# baseline (speedup 1.0000x reference)
import jax
import jax.numpy as jnp
from jax.experimental import pallas as pl


_ROW_BLOCK = 256


def _pairwise_entropy_kernel(x1_ref, x2_ref, o_ref):
    x1 = x1_ref[...]
    x2 = x2_ref[...]
    a = jnp.concatenate([x1 * jnp.log(x1), x1], axis=1)
    b = jnp.concatenate([x2, x2 * jnp.log(x2)], axis=1)
    o_ref[...] = -jax.lax.dot_general(
        a, b, (((1,), (1,)), ((), ())), preferred_element_type=jnp.float32
    )


def kernel(x1, x2):
    n1 = x1.shape[2]
    n2 = x2.shape[2]
    k = x1.shape[3]
    x1f = x1.reshape(n1, k)
    x2f = x2.reshape(n2, k)
    out = pl.pallas_call(
        _pairwise_entropy_kernel,
        grid=(n1 // _ROW_BLOCK,),
        in_specs=[
            pl.BlockSpec((_ROW_BLOCK, k), lambda i: (i, jnp.int32(0))),
            pl.BlockSpec((n2, k), lambda i: (jnp.int32(0), jnp.int32(0))),
        ],
        out_specs=pl.BlockSpec((_ROW_BLOCK, n2), lambda i: (i, jnp.int32(0))),
        out_shape=jax.ShapeDtypeStruct((n1, n2), jnp.float32),
    )(x1f, x2f)
    return out.astype(jnp.float64)
